# Initial kernel scaffold; baseline (speedup 1.0000x reference)
import jax
import jax.numpy as jnp
from jax.experimental import pallas as pl


def kernel(x, edge_index, batch, W1, b1, W2, b2, W3, b3, W_lin, b_lin):
    raise NotImplementedError("write your pallas kernel here")


# speedup vs baseline: 3.4038x; 3.4038x over previous
import jax
import jax.numpy as jnp
from jax import lax
from jax.experimental import pallas as pl
from jax.experimental.pallas import tpu as pltpu
from jax.experimental.pallas import tpu_sc as plsc

N_NODES = 10000
IN_CH = 256
HID = 512
OUT_CH = 64
N_GRAPHS = 128

NP = 10240
NW = 32
B = 128
NBATCH = 40
EPT = B * NBATCH
EP = EPT * NW
CW = 64
NCHUNK = HID // CW
ROWS_PER_TILE = NP // 16

_mesh = plsc.VectorSubcoreMesh(core_axis_name="c", subcore_axis_name="s",
                               num_cores=2, num_subcores=16)


def _deg_body(dst_hbm, ones_hbm, zeros_hbm, degp_hbm, idx_v, ones_v, shared):
    c = lax.axis_index("c")
    s = lax.axis_index("s")
    wid = c * 16 + s
    row0 = s * ROWS_PER_TILE
    pltpu.sync_copy(dst_hbm.at[wid], idx_v)
    pltpu.sync_copy(ones_hbm, ones_v)
    pltpu.sync_copy(zeros_hbm.at[pl.ds(row0, ROWS_PER_TILE)],
                    shared.at[pl.ds(row0, ROWS_PER_TILE)])
    plsc.subcore_barrier()

    def body(b, carry):
        pltpu.sync_copy(ones_v, shared.at[idx_v.at[b]], add=True)
        return carry

    lax.fori_loop(0, NBATCH, body, 0)
    plsc.subcore_barrier()
    pltpu.sync_copy(shared.at[pl.ds(row0, ROWS_PER_TILE)],
                    degp_hbm.at[c, pl.ds(row0, ROWS_PER_TILE)])


def _deg_call(dst_t, ones128, zeros128):
    return pl.kernel(
        _deg_body,
        out_type=jax.ShapeDtypeStruct((2, NP, 128), jnp.float32),
        mesh=_mesh,
        scratch_types=[
            pltpu.VMEM((NBATCH, B), jnp.int32),
            pltpu.VMEM((B, 128), jnp.float32),
            pltpu.VMEM_SHARED((NP, 128), jnp.float32),
        ],
    )(dst_t, ones128, zeros128)


NBUF = 8
NG = NBATCH // NBUF
assert NBATCH % NBUF == 0


def _agg_body(y0, y1, y2, y3, y4, y5, y6, y7, src_hbm, dst_hbm, zeros_hbm,
              out_hbm, si_v, di_v, rows_v, gsem, ssem, shared):
    c = lax.axis_index("c")
    s = lax.axis_index("s")
    wid = c * 16 + s
    row0 = s * ROWS_PER_TILE
    pltpu.sync_copy(src_hbm.at[wid], si_v)
    pltpu.sync_copy(dst_hbm.at[wid], di_v)
    for ci, y_hbm in enumerate((y0, y1, y2, y3, y4, y5, y6, y7)):
        pltpu.sync_copy(zeros_hbm.at[pl.ds(row0, ROWS_PER_TILE)],
                        shared.at[pl.ds(row0, ROWS_PER_TILE)])
        plsc.subcore_barrier()

        for k in range(NBUF):
            pltpu.async_copy(y_hbm.at[si_v.at[k]], rows_v.at[k], gsem.at[k])

        def group(g, carry, y_hbm=y_hbm):
            for k in range(NBUF):
                b = g * NBUF + k
                pltpu.make_async_copy(
                    y_hbm.at[si_v.at[b]], rows_v.at[k], gsem.at[k]).wait()
                pltpu.async_copy(rows_v.at[k], shared.at[di_v.at[b]],
                                 ssem.at[k], add=True)
            for k in range(NBUF):
                b = g * NBUF + k
                pltpu.make_async_copy(
                    rows_v.at[k], shared.at[di_v.at[b]], ssem.at[k]).wait()
                pltpu.async_copy(y_hbm.at[si_v.at[b + NBUF]], rows_v.at[k],
                                 gsem.at[k])
            return carry

        lax.fori_loop(0, NG - 1, group, 0)
        for k in range(NBUF):
            b = (NG - 1) * NBUF + k
            pltpu.make_async_copy(
                y_hbm.at[si_v.at[b]], rows_v.at[k], gsem.at[k]).wait()
            pltpu.async_copy(rows_v.at[k], shared.at[di_v.at[b]],
                             ssem.at[k], add=True)
        for k in range(NBUF):
            b = (NG - 1) * NBUF + k
            pltpu.make_async_copy(
                rows_v.at[k], shared.at[di_v.at[b]], ssem.at[k]).wait()
        plsc.subcore_barrier()
        pltpu.sync_copy(shared.at[pl.ds(row0, ROWS_PER_TILE)],
                        out_hbm.at[c, ci, pl.ds(row0, ROWS_PER_TILE)])
        plsc.subcore_barrier()


def _agg_call(yc, src_t, dst_t, zeros64):
    return pl.kernel(
        _agg_body,
        out_type=jax.ShapeDtypeStruct((2, NCHUNK, NP, CW), jnp.float32),
        mesh=_mesh,
        compiler_params=pltpu.CompilerParams(use_tc_tiling_on_sc=False),
        scratch_types=[
            pltpu.VMEM((NBATCH, B), jnp.int32),
            pltpu.VMEM((NBATCH, B), jnp.int32),
            pltpu.VMEM((NBUF, B, CW), jnp.float32),
            pltpu.SemaphoreType.DMA((NBUF,)),
            pltpu.SemaphoreType.DMA((NBUF,)),
            pltpu.VMEM_SHARED((NP, CW), jnp.float32),
        ],
    )(yc[0], yc[1], yc[2], yc[3], yc[4], yc[5], yc[6], yc[7],
      src_t, dst_t, zeros64)


def _stats_kernel(p0_ref, p1_ref, dis_ref):
    deg = 1.0 + p0_ref[:, :1] + p1_ref[:, :1]
    dis = lax.rsqrt(deg)
    dis_ref[...] = jnp.broadcast_to(dis, dis_ref.shape)


def _stats_call(degp):
    return pl.pallas_call(
        _stats_kernel,
        grid=(10,),
        in_specs=[
            pl.BlockSpec((1024, 128), lambda r: (r, 0)),
            pl.BlockSpec((1024, 128), lambda r: (r, 0)),
        ],
        out_specs=pl.BlockSpec((1024, CW), lambda r: (r, 0)),
        out_shape=jax.ShapeDtypeStruct((NP, CW), jnp.float32),
    )(degp[0], degp[1])


def _mm1_kernel(x_ref, w_ref, dis_ref, y_ref):
    xw = jnp.dot(x_ref[...], w_ref[...], preferred_element_type=jnp.float32)
    dis = dis_ref[...]
    for ci in range(NCHUNK):
        y_ref[ci] = dis * xw[:, ci * CW:(ci + 1) * CW]


def _mm1_call(x_pad, W1, dis):
    return pl.pallas_call(
        _mm1_kernel,
        grid=(20,),
        in_specs=[
            pl.BlockSpec((512, IN_CH), lambda r: (r, 0)),
            pl.BlockSpec((IN_CH, HID), lambda r: (0, 0)),
            pl.BlockSpec((512, CW), lambda r: (r, 0)),
        ],
        out_specs=pl.BlockSpec((NCHUNK, 512, CW), lambda r: (0, r, 0)),
        out_shape=jax.ShapeDtypeStruct((NCHUNK, NP, CW), jnp.float32),
    )(x_pad, W1, dis)


def _combine_mm_kernel(p_ref, y_ref, dis_ref, b_ref, w_ref, out_ref):
    dis = dis_ref[...]
    hs = []
    for ci in range(NCHUNK):
        pc = p_ref[0, ci] + p_ref[1, ci] + y_ref[ci]
        hs.append(jnp.maximum(dis * pc + b_ref[ci], 0.0))
    h = jnp.concatenate(hs, axis=1)
    xw = jnp.dot(h, w_ref[...], preferred_element_type=jnp.float32)
    for ci in range(NCHUNK):
        out_ref[ci] = dis * xw[:, ci * CW:(ci + 1) * CW]


def _combine_mm_call(part, y, dis, b_r, W):
    return pl.pallas_call(
        _combine_mm_kernel,
        grid=(20,),
        in_specs=[
            pl.BlockSpec((2, NCHUNK, 512, CW), lambda r: (0, 0, r, 0)),
            pl.BlockSpec((NCHUNK, 512, CW), lambda r: (0, r, 0)),
            pl.BlockSpec((512, CW), lambda r: (r, 0)),
            pl.BlockSpec((NCHUNK, 1, CW), lambda r: (0, 0, 0)),
            pl.BlockSpec((HID, HID), lambda r: (0, 0)),
        ],
        out_specs=pl.BlockSpec((NCHUNK, 512, CW), lambda r: (0, r, 0)),
        out_shape=jax.ShapeDtypeStruct((NCHUNK, NP, CW), jnp.float32),
    )(part, y, dis, b_r, W)


def _pool_kernel(p_ref, y_ref, dis_ref, b_ref, batch_ref, s_ref, cnt_ref):
    r = pl.program_id(0)
    dis = dis_ref[...]
    hs = []
    for ci in range(NCHUNK):
        pc = p_ref[0, ci] + p_ref[1, ci] + y_ref[ci]
        hs.append(jnp.maximum(dis * pc + b_ref[ci], 0.0))
    h = jnp.concatenate(hs, axis=1)
    iota = lax.broadcasted_iota(jnp.int32, (512, N_GRAPHS), 1)
    oh = (batch_ref[...] == iota).astype(jnp.float32)

    @pl.when(r == 0)
    def _():
        s_ref[...] = jnp.zeros_like(s_ref)
        cnt_ref[...] = jnp.zeros_like(cnt_ref)

    sb = lax.dot_general(h, oh, (((0,), (0,)), ((), ())),
                         preferred_element_type=jnp.float32)
    for ci in range(NCHUNK):
        s_ref[ci] += sb[ci * CW:(ci + 1) * CW, :]
    cnt_ref[0:1, :] += jnp.sum(oh, axis=0, keepdims=True)


def _pool_call(part, y, dis, b_r, batch_col):
    return pl.pallas_call(
        _pool_kernel,
        grid=(20,),
        in_specs=[
            pl.BlockSpec((2, NCHUNK, 512, CW), lambda r: (0, 0, r, 0)),
            pl.BlockSpec((NCHUNK, 512, CW), lambda r: (0, r, 0)),
            pl.BlockSpec((512, CW), lambda r: (r, 0)),
            pl.BlockSpec((NCHUNK, 1, CW), lambda r: (0, 0, 0)),
            pl.BlockSpec((512, 1), lambda r: (r, 0)),
        ],
        out_specs=[
            pl.BlockSpec((NCHUNK, CW, 128), lambda r: (0, 0, 0)),
            pl.BlockSpec((8, 128), lambda r: (0, 0)),
        ],
        out_shape=[
            jax.ShapeDtypeStruct((NCHUNK, CW, N_GRAPHS), jnp.float32),
            jax.ShapeDtypeStruct((8, N_GRAPHS), jnp.float32),
        ],
    )(part, y, dis, b_r, batch_col)


def _final_kernel(s_ref, w_ref, cnt_ref, b_ref, out_ref):
    acc = jnp.zeros((N_GRAPHS, OUT_CH), jnp.float32)
    for ci in range(NCHUNK):
        acc += lax.dot_general(s_ref[ci], w_ref[ci], (((0,), (0,)), ((), ())),
                               preferred_element_type=jnp.float32)
    inv = 1.0 / jnp.maximum(cnt_ref[...], 1.0)
    out_ref[...] = acc * inv + b_ref[...]


def _final_call(S, Wl_r, cnt_col, bl_row):
    return pl.pallas_call(
        _final_kernel,
        grid=(1,),
        in_specs=[
            pl.BlockSpec((NCHUNK, CW, N_GRAPHS), lambda i: (0, 0, 0)),
            pl.BlockSpec((NCHUNK, CW, OUT_CH), lambda i: (0, 0, 0)),
            pl.BlockSpec((N_GRAPHS, 1), lambda i: (0, 0)),
            pl.BlockSpec((1, OUT_CH), lambda i: (0, 0)),
        ],
        out_specs=pl.BlockSpec((N_GRAPHS, OUT_CH), lambda i: (0, 0)),
        out_shape=jax.ShapeDtypeStruct((N_GRAPHS, OUT_CH), jnp.float32),
    )(S, Wl_r, cnt_col, bl_row)


def kernel(x, edge_index, batch, W1, b1, W2, b2, W3, b3, W_lin, b_lin):
    src = edge_index[0].astype(jnp.int32)
    dst = edge_index[1].astype(jnp.int32)
    n_e = src.shape[0]

    src_t = jnp.concatenate(
        [src, jnp.zeros((EP - n_e,), jnp.int32)]).reshape(NW, NBATCH, B)
    dst_t = jnp.concatenate(
        [dst, jnp.full((EP - n_e,), N_NODES, jnp.int32)]).reshape(NW, NBATCH, B)
    x_pad = jnp.pad(x, ((0, NP - N_NODES), (0, 0)))
    batch_col = jnp.pad(batch.astype(jnp.int32), (0, NP - N_NODES),
                        constant_values=N_GRAPHS).reshape(NP, 1)
    ones128 = jnp.ones((B, 128), jnp.float32)
    zeros128 = jnp.zeros((NP, 128), jnp.float32)
    zeros64 = jnp.zeros((NP, CW), jnp.float32)
    b1_r = b1.reshape(NCHUNK, 1, CW)
    b2_r = b2.reshape(NCHUNK, 1, CW)
    b3_r = b3.reshape(NCHUNK, 1, CW)
    Wl_r = W_lin.reshape(NCHUNK, CW, OUT_CH)
    bl_row = b_lin.reshape(1, OUT_CH)

    degp = _deg_call(dst_t, ones128, zeros128)
    dis = _stats_call(degp)

    y = _mm1_call(x_pad, W1, dis)
    yc = [y[i] for i in range(NCHUNK)]
    part = _agg_call(yc, src_t, dst_t, zeros64)

    y = _combine_mm_call(part, y, dis, b1_r, W2)
    yc = [y[i] for i in range(NCHUNK)]
    part = _agg_call(yc, src_t, dst_t, zeros64)

    y = _combine_mm_call(part, y, dis, b2_r, W3)
    yc = [y[i] for i in range(NCHUNK)]
    part = _agg_call(yc, src_t, dst_t, zeros64)

    S, cnt = _pool_call(part, y, dis, b3_r, batch_col)
    cnt_col = cnt[0:1, :].reshape(N_GRAPHS, 1)
    return _final_call(S, Wl_r, cnt_col, bl_row)


# speedup vs baseline: 5.6777x; 1.6680x over previous
import jax
import jax.numpy as jnp
from jax import lax
from jax.experimental import pallas as pl
from jax.experimental.pallas import tpu as pltpu
from jax.experimental.pallas import tpu_sc as plsc

N_NODES = 10000
IN_CH = 256
HID = 512
OUT_CH = 64
N_GRAPHS = 128

NP = 10240
NW = 32
B = 128
NBATCH = 40
EPT = B * NBATCH
EP = EPT * NW
NT = 2
TW = HID // NT
ROWS_PER_TILE = NP // 16
PC = 4

_mesh = plsc.VectorSubcoreMesh(core_axis_name="c", subcore_axis_name="s",
                               num_cores=2, num_subcores=16)


def _deg_body(dst_hbm, ones_hbm, zeros_hbm, degp_hbm, idx_v, ones_v, shared):
    c = lax.axis_index("c")
    s = lax.axis_index("s")
    wid = c * 16 + s
    row0 = s * ROWS_PER_TILE
    pltpu.sync_copy(dst_hbm.at[wid], idx_v)
    pltpu.sync_copy(ones_hbm, ones_v)
    pltpu.sync_copy(zeros_hbm.at[pl.ds(row0, ROWS_PER_TILE)],
                    shared.at[pl.ds(row0, ROWS_PER_TILE)])
    plsc.subcore_barrier()

    def body(b, carry):
        pltpu.sync_copy(ones_v, shared.at[idx_v.at[b]], add=True)
        return carry

    lax.fori_loop(0, NBATCH, body, 0)
    plsc.subcore_barrier()
    pltpu.sync_copy(shared.at[pl.ds(row0, ROWS_PER_TILE)],
                    degp_hbm.at[c, pl.ds(row0, ROWS_PER_TILE)])


def _deg_call(dst_t, ones128, zeros128):
    return pl.kernel(
        _deg_body,
        out_type=jax.ShapeDtypeStruct((2, NP, 128), jnp.float32),
        mesh=_mesh,
        scratch_types=[
            pltpu.VMEM((NBATCH, B), jnp.int32),
            pltpu.VMEM((B, 128), jnp.float32),
            pltpu.VMEM_SHARED((NP, 128), jnp.float32),
        ],
    )(dst_t, ones128, zeros128)


NBUF = 2
HALF = NBATCH // 2
NGH = HALF // NBUF
assert HALF % NBUF == 0


def _agg_body(y0, y1, src_hbm, dst_hbm, zeros_hbm, out_hbm,
              si_v, di_v, rows_v, gsem, ssem, shared):
    c = lax.axis_index("c")
    s = lax.axis_index("s")
    wid = c * 16 + s
    row0 = s * ROWS_PER_TILE
    for t, y_hbm in enumerate((y0, y1)):
        pltpu.sync_copy(zeros_hbm.at[pl.ds(row0, ROWS_PER_TILE)],
                        shared.at[pl.ds(row0, ROWS_PER_TILE)])
        plsc.subcore_barrier()

        for h in range(2):
            pltpu.sync_copy(src_hbm.at[wid, pl.ds(h * HALF, HALF)], si_v)
            pltpu.sync_copy(dst_hbm.at[wid, pl.ds(h * HALF, HALF)], di_v)
            for k in range(NBUF):
                pltpu.async_copy(y_hbm.at[si_v.at[k]], rows_v.at[k],
                                 gsem.at[k])

            def group(g, carry, y_hbm=y_hbm):
                for k in range(NBUF):
                    b = g * NBUF + k
                    pltpu.make_async_copy(
                        y_hbm.at[si_v.at[b]], rows_v.at[k], gsem.at[k]).wait()
                    pltpu.async_copy(rows_v.at[k], shared.at[di_v.at[b]],
                                     ssem.at[k], add=True)
                for k in range(NBUF):
                    b = g * NBUF + k
                    pltpu.make_async_copy(
                        rows_v.at[k], shared.at[di_v.at[b]], ssem.at[k]).wait()
                    pltpu.async_copy(y_hbm.at[si_v.at[b + NBUF]],
                                     rows_v.at[k], gsem.at[k])
                return carry

            lax.fori_loop(0, NGH - 1, group, 0)
            for k in range(NBUF):
                b = (NGH - 1) * NBUF + k
                pltpu.make_async_copy(
                    y_hbm.at[si_v.at[b]], rows_v.at[k], gsem.at[k]).wait()
                pltpu.async_copy(rows_v.at[k], shared.at[di_v.at[b]],
                                 ssem.at[k], add=True)
            for k in range(NBUF):
                b = (NGH - 1) * NBUF + k
                pltpu.make_async_copy(
                    rows_v.at[k], shared.at[di_v.at[b]], ssem.at[k]).wait()
        plsc.subcore_barrier()
        pltpu.sync_copy(shared.at[pl.ds(row0, ROWS_PER_TILE)],
                        out_hbm.at[c, t, pl.ds(row0, ROWS_PER_TILE)])
        plsc.subcore_barrier()


def _agg_call(yc, src_t, dst_t, zeros_b):
    return pl.kernel(
        _agg_body,
        out_type=jax.ShapeDtypeStruct((2, NT, NP, 2, 128), jnp.bfloat16),
        mesh=_mesh,
        compiler_params=pltpu.CompilerParams(use_tc_tiling_on_sc=False),
        scratch_types=[
            pltpu.VMEM((HALF, B), jnp.int32),
            pltpu.VMEM((HALF, B), jnp.int32),
            pltpu.VMEM((NBUF, B, 2, 128), jnp.bfloat16),
            pltpu.SemaphoreType.DMA((NBUF,)),
            pltpu.SemaphoreType.DMA((NBUF,)),
            pltpu.VMEM_SHARED((NP, 2, 128), jnp.bfloat16),
        ],
    )(yc[0], yc[1], src_t, dst_t, zeros_b)


def _stats_kernel(p0_ref, p1_ref, dis_ref):
    deg = 1.0 + p0_ref[:, :1] + p1_ref[:, :1]
    dis = lax.rsqrt(deg)
    dis_ref[...] = jnp.broadcast_to(dis, dis_ref.shape)


def _stats_call(degp):
    return pl.pallas_call(
        _stats_kernel,
        grid=(10,),
        in_specs=[
            pl.BlockSpec((1024, 128), lambda r: (r, 0)),
            pl.BlockSpec((1024, 128), lambda r: (r, 0)),
        ],
        out_specs=pl.BlockSpec((1024, 128), lambda r: (r, 0)),
        out_shape=jax.ShapeDtypeStruct((NP, 128), jnp.float32),
    )(degp[0], degp[1])


def _to_tables(xw, dis, y_ref):
    d = jnp.broadcast_to(dis[:, :1], (512, TW))
    for t in range(NT):
        yt = d * xw[:, t * TW:(t + 1) * TW]
        y_ref[t] = yt.astype(jnp.bfloat16)


def _mm1_kernel(x_ref, w_ref, dis_ref, y_ref):
    xw = jnp.dot(x_ref[...], w_ref[...], preferred_element_type=jnp.float32)
    _to_tables(xw, dis_ref[...], y_ref)


def _mm1_call(x_pad, W1, dis):
    return pl.pallas_call(
        _mm1_kernel,
        grid=(20,),
        in_specs=[
            pl.BlockSpec((512, IN_CH), lambda r: (r, 0)),
            pl.BlockSpec((IN_CH, HID), lambda r: (0, 0)),
            pl.BlockSpec((512, 128), lambda r: (r, 0)),
        ],
        out_specs=pl.BlockSpec((NT, 512, TW), lambda r: (0, r, 0)),
        out_shape=jax.ShapeDtypeStruct((NT, NP, TW), jnp.bfloat16),
    )(x_pad, W1, dis)


def _combine(p_ref, y_ref, dis, b_ref):
    hs = []
    d = jnp.broadcast_to(dis[:, :1], (512, TW))
    for t in range(NT):
        pc = (p_ref[0, t].astype(jnp.float32)
              + p_ref[1, t].astype(jnp.float32)
              + y_ref[t].astype(jnp.float32))
        hs.append(jnp.maximum(d * pc + b_ref[t], 0.0))
    return jnp.concatenate(hs, axis=1)


def _combine_mm_kernel(p_ref, y_ref, dis_ref, b_ref, w_ref, out_ref):
    h = _combine(p_ref, y_ref, dis_ref[...], b_ref)
    xw = jnp.dot(h, w_ref[...], preferred_element_type=jnp.float32)
    _to_tables(xw, dis_ref[...], out_ref)


def _combine_mm_call(part, y, dis, b_r, W):
    return pl.pallas_call(
        _combine_mm_kernel,
        grid=(20,),
        in_specs=[
            pl.BlockSpec((2, NT, 512, TW), lambda r: (0, 0, r, 0)),
            pl.BlockSpec((NT, 512, TW), lambda r: (0, r, 0)),
            pl.BlockSpec((512, 128), lambda r: (r, 0)),
            pl.BlockSpec((NT, 1, TW), lambda r: (0, 0, 0)),
            pl.BlockSpec((HID, HID), lambda r: (0, 0)),
        ],
        out_specs=pl.BlockSpec((NT, 512, TW), lambda r: (0, r, 0)),
        out_shape=jax.ShapeDtypeStruct((NT, NP, TW), jnp.bfloat16),
    )(part, y, dis, b_r, W)


def _pool_kernel(p_ref, y_ref, dis_ref, b_ref, batch_ref, s_ref, cnt_ref):
    r = pl.program_id(0)
    h = _combine(p_ref, y_ref, dis_ref[...], b_ref)
    iota = lax.broadcasted_iota(jnp.int32, (512, N_GRAPHS), 1)
    oh = (batch_ref[...] == iota).astype(jnp.float32)

    @pl.when(r == 0)
    def _():
        s_ref[...] = jnp.zeros_like(s_ref)
        cnt_ref[...] = jnp.zeros_like(cnt_ref)

    sb = lax.dot_general(h, oh, (((0,), (0,)), ((), ())),
                         preferred_element_type=jnp.float32)
    for ci in range(PC):
        s_ref[ci] += sb[ci * 128:(ci + 1) * 128, :]
    cnt_ref[0:1, :] += jnp.sum(oh, axis=0, keepdims=True)


def _pool_call(part, y, dis, b_r, batch_col):
    return pl.pallas_call(
        _pool_kernel,
        grid=(20,),
        in_specs=[
            pl.BlockSpec((2, NT, 512, TW), lambda r: (0, 0, r, 0)),
            pl.BlockSpec((NT, 512, TW), lambda r: (0, r, 0)),
            pl.BlockSpec((512, 128), lambda r: (r, 0)),
            pl.BlockSpec((NT, 1, TW), lambda r: (0, 0, 0)),
            pl.BlockSpec((512, 1), lambda r: (r, 0)),
        ],
        out_specs=[
            pl.BlockSpec((PC, 128, 128), lambda r: (0, 0, 0)),
            pl.BlockSpec((8, 128), lambda r: (0, 0)),
        ],
        out_shape=[
            jax.ShapeDtypeStruct((PC, 128, N_GRAPHS), jnp.float32),
            jax.ShapeDtypeStruct((8, N_GRAPHS), jnp.float32),
        ],
    )(part, y, dis, b_r, batch_col)


def _final_kernel(s_ref, w_ref, cnt_ref, b_ref, out_ref):
    acc = jnp.zeros((N_GRAPHS, OUT_CH), jnp.float32)
    for ci in range(PC):
        acc += lax.dot_general(s_ref[ci], w_ref[ci], (((0,), (0,)), ((), ())),
                               preferred_element_type=jnp.float32)
    inv = 1.0 / jnp.maximum(cnt_ref[...], 1.0)
    out_ref[...] = acc * inv + b_ref[...]


def _final_call(S, Wl_r, cnt_col, bl_row):
    return pl.pallas_call(
        _final_kernel,
        grid=(1,),
        in_specs=[
            pl.BlockSpec((PC, 128, N_GRAPHS), lambda i: (0, 0, 0)),
            pl.BlockSpec((PC, 128, OUT_CH), lambda i: (0, 0, 0)),
            pl.BlockSpec((N_GRAPHS, 1), lambda i: (0, 0)),
            pl.BlockSpec((1, OUT_CH), lambda i: (0, 0)),
        ],
        out_specs=pl.BlockSpec((N_GRAPHS, OUT_CH), lambda i: (0, 0)),
        out_shape=jax.ShapeDtypeStruct((N_GRAPHS, OUT_CH), jnp.float32),
    )(S, Wl_r, cnt_col, bl_row)


def kernel(x, edge_index, batch, W1, b1, W2, b2, W3, b3, W_lin, b_lin):
    src = edge_index[0].astype(jnp.int32)
    dst = edge_index[1].astype(jnp.int32)
    n_e = src.shape[0]

    src_t = jnp.concatenate(
        [src, jnp.zeros((EP - n_e,), jnp.int32)]).reshape(NW, NBATCH, B)
    dst_t = jnp.concatenate(
        [dst, jnp.full((EP - n_e,), N_NODES, jnp.int32)]).reshape(NW, NBATCH, B)
    x_pad = jnp.pad(x, ((0, NP - N_NODES), (0, 0)))
    batch_col = jnp.pad(batch.astype(jnp.int32), (0, NP - N_NODES),
                        constant_values=N_GRAPHS).reshape(NP, 1)
    ones128 = jnp.ones((B, 128), jnp.float32)
    zeros128 = jnp.zeros((NP, 128), jnp.float32)
    zeros_b = jnp.zeros((NP, 2, 128), jnp.bfloat16)
    b1_r = b1.reshape(NT, 1, TW)
    b2_r = b2.reshape(NT, 1, TW)
    b3_r = b3.reshape(NT, 1, TW)
    Wl_r = W_lin.reshape(PC, 128, OUT_CH)
    bl_row = b_lin.reshape(1, OUT_CH)

    degp = _deg_call(dst_t, ones128, zeros128)
    dis = _stats_call(degp)

    def to_sc(y2d):
        y3 = y2d.reshape(NT, NP, 2, 128)
        return [y3[0], y3[1]]

    y = _mm1_call(x_pad, W1, dis)
    part = _agg_call(to_sc(y), src_t, dst_t, zeros_b).reshape(2, NT, NP, TW)

    y = _combine_mm_call(part, y, dis, b1_r, W2)
    part = _agg_call(to_sc(y), src_t, dst_t, zeros_b).reshape(2, NT, NP, TW)

    y = _combine_mm_call(part, y, dis, b2_r, W3)
    part = _agg_call(to_sc(y), src_t, dst_t, zeros_b).reshape(2, NT, NP, TW)

    S, cnt = _pool_call(part, y, dis, b3_r, batch_col)
    cnt_col = cnt[0:1, :].reshape(N_GRAPHS, 1)
    return _final_call(S, Wl_r, cnt_col, bl_row)


# speedup vs baseline: 11.2222x; 1.9766x over previous
import jax
import jax.numpy as jnp
from jax import lax
from jax.experimental import pallas as pl
from jax.experimental.pallas import tpu as pltpu
from jax.experimental.pallas import tpu_sc as plsc

N_NODES = 10000
IN_CH = 256
HID = 512
OUT_CH = 64
N_GRAPHS = 128

NP = 10240
NW = 32
B = 128
NBATCH = 40
EPT = B * NBATCH
EP = EPT * NW
NT = 2
TW = HID // NT
ROWS_PER_TILE = NP // 16
PC = 4

_mesh = plsc.VectorSubcoreMesh(core_axis_name="c", subcore_axis_name="s",
                               num_cores=2, num_subcores=16)


def _deg_body(dst_hbm, ones_hbm, zeros_hbm, degp_hbm, idx_v, ones_v, shared):
    c = lax.axis_index("c")
    s = lax.axis_index("s")
    wid = c * 16 + s
    row0 = s * ROWS_PER_TILE
    pltpu.sync_copy(dst_hbm.at[wid], idx_v)
    pltpu.sync_copy(ones_hbm, ones_v)
    pltpu.sync_copy(zeros_hbm.at[pl.ds(row0, ROWS_PER_TILE)],
                    shared.at[pl.ds(row0, ROWS_PER_TILE)])
    plsc.subcore_barrier()

    def body(b, carry):
        pltpu.sync_copy(ones_v, shared.at[idx_v.at[b]], add=True)
        return carry

    lax.fori_loop(0, NBATCH, body, 0)
    plsc.subcore_barrier()
    pltpu.sync_copy(shared.at[pl.ds(row0, ROWS_PER_TILE)],
                    degp_hbm.at[c, pl.ds(row0, ROWS_PER_TILE)])


def _deg_call(dst_t, ones128, zeros128):
    return pl.kernel(
        _deg_body,
        out_type=jax.ShapeDtypeStruct((2, NP, 128), jnp.float32),
        mesh=_mesh,
        scratch_types=[
            pltpu.VMEM((NBATCH, B), jnp.int32),
            pltpu.VMEM((B, 128), jnp.float32),
            pltpu.VMEM_SHARED((NP, 128), jnp.float32),
        ],
    )(dst_t, ones128, zeros128)


NBUF = 2
HALF = NBATCH // 2
NGH = HALF // NBUF
assert HALF % NBUF == 0


def _agg_body(y0, y1, src_hbm, dst_hbm, zeros_hbm, out_hbm,
              si_v, di_v, rows_v, gsem, ssem, shared):
    c = lax.axis_index("c")
    s = lax.axis_index("s")
    wid = c * 16 + s
    row0 = s * ROWS_PER_TILE
    for t, y_hbm in enumerate((y0, y1)):
        pltpu.sync_copy(zeros_hbm.at[pl.ds(row0, ROWS_PER_TILE)],
                        shared.at[pl.ds(row0, ROWS_PER_TILE)])
        plsc.subcore_barrier()

        for h in range(2):
            pltpu.sync_copy(src_hbm.at[wid, pl.ds(h * HALF, HALF)], si_v)
            pltpu.sync_copy(dst_hbm.at[wid, pl.ds(h * HALF, HALF)], di_v)
            for k in range(NBUF):
                pltpu.async_copy(y_hbm.at[si_v.at[k]], rows_v.at[k],
                                 gsem.at[k])

            def group(g, carry, y_hbm=y_hbm):
                for k in range(NBUF):
                    b = g * NBUF + k
                    pltpu.make_async_copy(
                        y_hbm.at[si_v.at[b]], rows_v.at[k], gsem.at[k]).wait()
                    pltpu.async_copy(rows_v.at[k], shared.at[di_v.at[b]],
                                     ssem.at[k], add=True)
                for k in range(NBUF):
                    b = g * NBUF + k
                    pltpu.make_async_copy(
                        rows_v.at[k], shared.at[di_v.at[b]], ssem.at[k]).wait()
                    pltpu.async_copy(y_hbm.at[si_v.at[b + NBUF]],
                                     rows_v.at[k], gsem.at[k])
                return carry

            lax.fori_loop(0, NGH - 1, group, 0)
            for k in range(NBUF):
                b = (NGH - 1) * NBUF + k
                pltpu.make_async_copy(
                    y_hbm.at[si_v.at[b]], rows_v.at[k], gsem.at[k]).wait()
                pltpu.async_copy(rows_v.at[k], shared.at[di_v.at[b]],
                                 ssem.at[k], add=True)
            for k in range(NBUF):
                b = (NGH - 1) * NBUF + k
                pltpu.make_async_copy(
                    rows_v.at[k], shared.at[di_v.at[b]], ssem.at[k]).wait()
        plsc.subcore_barrier()
        pltpu.sync_copy(shared.at[pl.ds(row0, ROWS_PER_TILE)],
                        out_hbm.at[c, t, pl.ds(row0, ROWS_PER_TILE)])
        plsc.subcore_barrier()


def _agg_call(yc, src_t, dst_t, zeros_b):
    return pl.kernel(
        _agg_body,
        out_type=jax.ShapeDtypeStruct((2, NT, NP, 2, 128), jnp.bfloat16),
        mesh=_mesh,
        compiler_params=pltpu.CompilerParams(use_tc_tiling_on_sc=False),
        scratch_types=[
            pltpu.VMEM((HALF, B), jnp.int32),
            pltpu.VMEM((HALF, B), jnp.int32),
            pltpu.VMEM((NBUF, B, 2, 128), jnp.bfloat16),
            pltpu.SemaphoreType.DMA((NBUF,)),
            pltpu.SemaphoreType.DMA((NBUF,)),
            pltpu.VMEM_SHARED((NP, 2, 128), jnp.bfloat16),
        ],
    )(yc[0], yc[1], src_t, dst_t, zeros_b)


def _stats_kernel(p0_ref, p1_ref, dis_ref):
    deg = 1.0 + p0_ref[:, :1] + p1_ref[:, :1]
    dis = lax.rsqrt(deg)
    dis_ref[...] = jnp.broadcast_to(dis, dis_ref.shape)


def _stats_call(degp):
    return pl.pallas_call(
        _stats_kernel,
        grid=(10,),
        in_specs=[
            pl.BlockSpec((1024, 128), lambda r: (r, 0)),
            pl.BlockSpec((1024, 128), lambda r: (r, 0)),
        ],
        out_specs=pl.BlockSpec((1024, 128), lambda r: (r, 0)),
        out_shape=jax.ShapeDtypeStruct((NP, 128), jnp.float32),
    )(degp[0], degp[1])


def _to_tables(xw, dis, y_ref):
    d = jnp.broadcast_to(dis[:, :1], (512, TW))
    for t in range(NT):
        yt = d * xw[:, t * TW:(t + 1) * TW]
        y_ref[t] = yt.astype(jnp.bfloat16)


def _mm1_kernel(x_ref, w_ref, dis_ref, y_ref):
    xw = jnp.dot(x_ref[...], w_ref[...], preferred_element_type=jnp.float32)
    _to_tables(xw, dis_ref[...], y_ref)


def _mm1_call(x_pad, W1, dis):
    return pl.pallas_call(
        _mm1_kernel,
        grid=(20,),
        in_specs=[
            pl.BlockSpec((512, IN_CH), lambda r: (r, 0)),
            pl.BlockSpec((IN_CH, HID), lambda r: (0, 0)),
            pl.BlockSpec((512, 128), lambda r: (r, 0)),
        ],
        out_specs=pl.BlockSpec((NT, 512, TW), lambda r: (0, r, 0)),
        out_shape=jax.ShapeDtypeStruct((NT, NP, TW), jnp.bfloat16),
    )(x_pad, W1, dis)


def _combine(p_ref, y_ref, dis, b_ref):
    hs = []
    d = jnp.broadcast_to(dis[:, :1], (512, TW))
    for t in range(NT):
        pc = (p_ref[0, t].astype(jnp.float32)
              + p_ref[1, t].astype(jnp.float32)
              + y_ref[t].astype(jnp.float32))
        hs.append(jnp.maximum(d * pc + b_ref[t], 0.0))
    return jnp.concatenate(hs, axis=1)


def _combine_mm_kernel(p_ref, y_ref, dis_ref, b_ref, w_ref, out_ref):
    h = _combine(p_ref, y_ref, dis_ref[...], b_ref)
    xw = jnp.dot(h, w_ref[...], preferred_element_type=jnp.float32)
    _to_tables(xw, dis_ref[...], out_ref)


def _combine_mm_call(part, y, dis, b_r, W):
    return pl.pallas_call(
        _combine_mm_kernel,
        grid=(20,),
        in_specs=[
            pl.BlockSpec((2, NT, 512, TW), lambda r: (0, 0, r, 0)),
            pl.BlockSpec((NT, 512, TW), lambda r: (0, r, 0)),
            pl.BlockSpec((512, 128), lambda r: (r, 0)),
            pl.BlockSpec((NT, 1, TW), lambda r: (0, 0, 0)),
            pl.BlockSpec((HID, HID), lambda r: (0, 0)),
        ],
        out_specs=pl.BlockSpec((NT, 512, TW), lambda r: (0, r, 0)),
        out_shape=jax.ShapeDtypeStruct((NT, NP, TW), jnp.bfloat16),
    )(part, y, dis, b_r, W)


def _pool_kernel(p_ref, y_ref, dis_ref, b_ref, batch_ref, s_ref, cnt_ref):
    r = pl.program_id(0)
    h = _combine(p_ref, y_ref, dis_ref[...], b_ref)
    iota = lax.broadcasted_iota(jnp.int32, (512, N_GRAPHS), 1)
    oh = (batch_ref[...] == iota).astype(jnp.float32)

    @pl.when(r == 0)
    def _():
        s_ref[...] = jnp.zeros_like(s_ref)
        cnt_ref[...] = jnp.zeros_like(cnt_ref)

    sb = lax.dot_general(h, oh, (((0,), (0,)), ((), ())),
                         preferred_element_type=jnp.float32)
    for ci in range(PC):
        s_ref[ci] += sb[ci * 128:(ci + 1) * 128, :]
    cnt_ref[0:1, :] += jnp.sum(oh, axis=0, keepdims=True)


def _pool_call(part, y, dis, b_r, batch_col):
    return pl.pallas_call(
        _pool_kernel,
        grid=(20,),
        in_specs=[
            pl.BlockSpec((2, NT, 512, TW), lambda r: (0, 0, r, 0)),
            pl.BlockSpec((NT, 512, TW), lambda r: (0, r, 0)),
            pl.BlockSpec((512, 128), lambda r: (r, 0)),
            pl.BlockSpec((NT, 1, TW), lambda r: (0, 0, 0)),
            pl.BlockSpec((512, 1), lambda r: (r, 0)),
        ],
        out_specs=[
            pl.BlockSpec((PC, 128, 128), lambda r: (0, 0, 0)),
            pl.BlockSpec((8, 128), lambda r: (0, 0)),
        ],
        out_shape=[
            jax.ShapeDtypeStruct((PC, 128, N_GRAPHS), jnp.float32),
            jax.ShapeDtypeStruct((8, N_GRAPHS), jnp.float32),
        ],
    )(part, y, dis, b_r, batch_col)


def _final_kernel(s_ref, w_ref, cnt_ref, b_ref, out_ref):
    acc = jnp.zeros((N_GRAPHS, OUT_CH), jnp.float32)
    for ci in range(PC):
        acc += lax.dot_general(s_ref[ci], w_ref[ci], (((0,), (0,)), ((), ())),
                               preferred_element_type=jnp.float32)
    inv = 1.0 / jnp.maximum(cnt_ref[...], 1.0)
    out_ref[...] = acc * inv + b_ref[...]


def _final_call(S, Wl_r, cnt_col, bl_row):
    return pl.pallas_call(
        _final_kernel,
        grid=(1,),
        in_specs=[
            pl.BlockSpec((PC, 128, N_GRAPHS), lambda i: (0, 0, 0)),
            pl.BlockSpec((PC, 128, OUT_CH), lambda i: (0, 0, 0)),
            pl.BlockSpec((N_GRAPHS, 1), lambda i: (0, 0)),
            pl.BlockSpec((1, OUT_CH), lambda i: (0, 0)),
        ],
        out_specs=pl.BlockSpec((N_GRAPHS, OUT_CH), lambda i: (0, 0)),
        out_shape=jax.ShapeDtypeStruct((N_GRAPHS, OUT_CH), jnp.float32),
    )(S, Wl_r, cnt_col, bl_row)


def kernel(x, edge_index, batch, W1, b1, W2, b2, W3, b3, W_lin, b_lin):
    src = edge_index[0].astype(jnp.int32)
    dst = edge_index[1].astype(jnp.int32)
    n_e = src.shape[0]

    dummy = N_NODES + (jnp.arange(EP - n_e, dtype=jnp.int32) % (NP - N_NODES))
    src_t = jnp.concatenate([src, dummy]).reshape(NW, NBATCH, B)
    dst_t = jnp.concatenate([dst, dummy]).reshape(NW, NBATCH, B)
    x_pad = jnp.pad(x, ((0, NP - N_NODES), (0, 0)))
    batch_col = jnp.pad(batch.astype(jnp.int32), (0, NP - N_NODES),
                        constant_values=N_GRAPHS).reshape(NP, 1)
    ones128 = jnp.ones((B, 128), jnp.float32)
    zeros128 = jnp.zeros((NP, 128), jnp.float32)
    zeros_b = jnp.zeros((NP, 2, 128), jnp.bfloat16)
    b1_r = b1.reshape(NT, 1, TW)
    b2_r = b2.reshape(NT, 1, TW)
    b3_r = b3.reshape(NT, 1, TW)
    Wl_r = W_lin.reshape(PC, 128, OUT_CH)
    bl_row = b_lin.reshape(1, OUT_CH)

    degp = _deg_call(dst_t, ones128, zeros128)
    dis = _stats_call(degp)

    def to_sc(y2d):
        y3 = y2d.reshape(NT, NP, 2, 128)
        return [y3[0], y3[1]]

    y = _mm1_call(x_pad, W1, dis)
    part = _agg_call(to_sc(y), src_t, dst_t, zeros_b).reshape(2, NT, NP, TW)

    y = _combine_mm_call(part, y, dis, b1_r, W2)
    part = _agg_call(to_sc(y), src_t, dst_t, zeros_b).reshape(2, NT, NP, TW)

    y = _combine_mm_call(part, y, dis, b2_r, W3)
    part = _agg_call(to_sc(y), src_t, dst_t, zeros_b).reshape(2, NT, NP, TW)

    S, cnt = _pool_call(part, y, dis, b3_r, batch_col)
    cnt_col = cnt[0:1, :].reshape(N_GRAPHS, 1)
    return _final_call(S, Wl_r, cnt_col, bl_row)


# speedup vs baseline: 11.4088x; 1.0166x over previous
import jax
import jax.numpy as jnp
from jax import lax
from jax.experimental import pallas as pl
from jax.experimental.pallas import tpu as pltpu
from jax.experimental.pallas import tpu_sc as plsc

N_NODES = 10000
IN_CH = 256
HID = 512
OUT_CH = 64
N_GRAPHS = 128

NP = 10240
NW = 32
B = 128
NBATCH = 40
EPT = B * NBATCH
EP = EPT * NW
NT = 2
TW = HID // NT
ROWS_PER_TILE = NP // 16
PC = 4

_mesh = plsc.VectorSubcoreMesh(core_axis_name="c", subcore_axis_name="s",
                               num_cores=2, num_subcores=16)


def _deg_body(dst_hbm, ones_hbm, zeros_hbm, degp_hbm, idx_v, ones_v, shared):
    c = lax.axis_index("c")
    s = lax.axis_index("s")
    wid = c * 16 + s
    row0 = s * ROWS_PER_TILE
    pltpu.sync_copy(dst_hbm.at[wid], idx_v)
    pltpu.sync_copy(ones_hbm, ones_v)
    pltpu.sync_copy(zeros_hbm.at[pl.ds(row0, ROWS_PER_TILE)],
                    shared.at[pl.ds(row0, ROWS_PER_TILE)])
    plsc.subcore_barrier()

    def body(b, carry):
        pltpu.sync_copy(ones_v, shared.at[idx_v.at[b]], add=True)
        return carry

    lax.fori_loop(0, NBATCH, body, 0)
    plsc.subcore_barrier()
    pltpu.sync_copy(shared.at[pl.ds(row0, ROWS_PER_TILE)],
                    degp_hbm.at[c, pl.ds(row0, ROWS_PER_TILE)])


def _deg_call(dst_t, ones128, zeros128):
    return pl.kernel(
        _deg_body,
        out_type=jax.ShapeDtypeStruct((2, NP, 128), jnp.float32),
        mesh=_mesh,
        scratch_types=[
            pltpu.VMEM((NBATCH, B), jnp.int32),
            pltpu.VMEM((B, 128), jnp.float32),
            pltpu.VMEM_SHARED((NP, 128), jnp.float32),
        ],
    )(dst_t, ones128, zeros128)


NBUF = 2
HALF = NBATCH
NGH = HALF // NBUF
assert HALF % NBUF == 0


def _agg_body(y0, y1, src_hbm, dst_hbm, zeros_hbm, out_hbm,
              si_v, di_v, rows_v, gsem, ssem, shared):
    c = lax.axis_index("c")
    s = lax.axis_index("s")
    wid = c * 16 + s
    row0 = s * ROWS_PER_TILE
    for t, y_hbm in enumerate((y0, y1)):
        pltpu.sync_copy(zeros_hbm.at[pl.ds(row0, ROWS_PER_TILE)],
                        shared.at[pl.ds(row0, ROWS_PER_TILE)])
        plsc.subcore_barrier()

        for h in range(1):
            pltpu.sync_copy(src_hbm.at[wid, pl.ds(h * HALF, HALF)], si_v)
            pltpu.sync_copy(dst_hbm.at[wid, pl.ds(h * HALF, HALF)], di_v)
            for k in range(NBUF):
                pltpu.async_copy(y_hbm.at[si_v.at[k]], rows_v.at[k],
                                 gsem.at[k])

            def group(g, carry, y_hbm=y_hbm):
                for k in range(NBUF):
                    b = g * NBUF + k
                    pltpu.make_async_copy(
                        y_hbm.at[si_v.at[b]], rows_v.at[k], gsem.at[k]).wait()
                    pltpu.async_copy(rows_v.at[k], shared.at[di_v.at[b]],
                                     ssem.at[k], add=True)
                for k in range(NBUF):
                    b = g * NBUF + k
                    pltpu.make_async_copy(
                        rows_v.at[k], shared.at[di_v.at[b]], ssem.at[k]).wait()
                    pltpu.async_copy(y_hbm.at[si_v.at[b + NBUF]],
                                     rows_v.at[k], gsem.at[k])
                return carry

            lax.fori_loop(0, NGH - 1, group, 0)
            for k in range(NBUF):
                b = (NGH - 1) * NBUF + k
                pltpu.make_async_copy(
                    y_hbm.at[si_v.at[b]], rows_v.at[k], gsem.at[k]).wait()
                pltpu.async_copy(rows_v.at[k], shared.at[di_v.at[b]],
                                 ssem.at[k], add=True)
            for k in range(NBUF):
                b = (NGH - 1) * NBUF + k
                pltpu.make_async_copy(
                    rows_v.at[k], shared.at[di_v.at[b]], ssem.at[k]).wait()
        plsc.subcore_barrier()
        pltpu.sync_copy(shared.at[pl.ds(row0, ROWS_PER_TILE)],
                        out_hbm.at[c, t, pl.ds(row0, ROWS_PER_TILE)])
        plsc.subcore_barrier()


def _agg_call(yc, src_t, dst_t, zeros_b):
    return pl.kernel(
        _agg_body,
        out_type=jax.ShapeDtypeStruct((2, NT, NP, 2, 128), jnp.bfloat16),
        mesh=_mesh,
        compiler_params=pltpu.CompilerParams(use_tc_tiling_on_sc=False),
        scratch_types=[
            pltpu.VMEM((HALF, B), jnp.int32),
            pltpu.VMEM((HALF, B), jnp.int32),
            pltpu.VMEM((NBUF, B, 2, 128), jnp.bfloat16),
            pltpu.SemaphoreType.DMA((NBUF,)),
            pltpu.SemaphoreType.DMA((NBUF,)),
            pltpu.VMEM_SHARED((NP, 2, 128), jnp.bfloat16),
        ],
    )(yc[0], yc[1], src_t, dst_t, zeros_b)


def _stats_kernel(p0_ref, p1_ref, dis_ref):
    deg = 1.0 + p0_ref[:, :1] + p1_ref[:, :1]
    dis = lax.rsqrt(deg)
    dis_ref[...] = jnp.broadcast_to(dis, dis_ref.shape)


def _stats_call(degp):
    return pl.pallas_call(
        _stats_kernel,
        grid=(10,),
        in_specs=[
            pl.BlockSpec((1024, 128), lambda r: (r, 0)),
            pl.BlockSpec((1024, 128), lambda r: (r, 0)),
        ],
        out_specs=pl.BlockSpec((1024, 128), lambda r: (r, 0)),
        out_shape=jax.ShapeDtypeStruct((NP, 128), jnp.float32),
    )(degp[0], degp[1])


def _to_tables(xw, dis, y_ref):
    d = jnp.broadcast_to(dis[:, :1], (512, TW))
    for t in range(NT):
        yt = d * xw[:, t * TW:(t + 1) * TW]
        y_ref[t] = yt.astype(jnp.bfloat16)


def _mm1_kernel(x_ref, w_ref, dis_ref, y_ref):
    xw = jnp.dot(x_ref[...], w_ref[...], preferred_element_type=jnp.float32)
    _to_tables(xw, dis_ref[...], y_ref)


def _mm1_call(x_pad, W1, dis):
    return pl.pallas_call(
        _mm1_kernel,
        grid=(20,),
        in_specs=[
            pl.BlockSpec((512, IN_CH), lambda r: (r, 0)),
            pl.BlockSpec((IN_CH, HID), lambda r: (0, 0)),
            pl.BlockSpec((512, 128), lambda r: (r, 0)),
        ],
        out_specs=pl.BlockSpec((NT, 512, TW), lambda r: (0, r, 0)),
        out_shape=jax.ShapeDtypeStruct((NT, NP, TW), jnp.bfloat16),
    )(x_pad, W1, dis)


def _combine(p_ref, y_ref, dis, b_ref):
    hs = []
    d = jnp.broadcast_to(dis[:, :1], (512, TW))
    for t in range(NT):
        pc = (p_ref[0, t].astype(jnp.float32)
              + p_ref[1, t].astype(jnp.float32)
              + y_ref[t].astype(jnp.float32))
        hs.append(jnp.maximum(d * pc + b_ref[t], 0.0))
    return jnp.concatenate(hs, axis=1)


def _combine_mm_kernel(p_ref, y_ref, dis_ref, b_ref, w_ref, out_ref):
    h = _combine(p_ref, y_ref, dis_ref[...], b_ref)
    xw = jnp.dot(h, w_ref[...], preferred_element_type=jnp.float32)
    _to_tables(xw, dis_ref[...], out_ref)


def _combine_mm_call(part, y, dis, b_r, W):
    return pl.pallas_call(
        _combine_mm_kernel,
        grid=(20,),
        in_specs=[
            pl.BlockSpec((2, NT, 512, TW), lambda r: (0, 0, r, 0)),
            pl.BlockSpec((NT, 512, TW), lambda r: (0, r, 0)),
            pl.BlockSpec((512, 128), lambda r: (r, 0)),
            pl.BlockSpec((NT, 1, TW), lambda r: (0, 0, 0)),
            pl.BlockSpec((HID, HID), lambda r: (0, 0)),
        ],
        out_specs=pl.BlockSpec((NT, 512, TW), lambda r: (0, r, 0)),
        out_shape=jax.ShapeDtypeStruct((NT, NP, TW), jnp.bfloat16),
    )(part, y, dis, b_r, W)


def _pool_kernel(p_ref, y_ref, dis_ref, b_ref, batch_ref, s_ref, cnt_ref):
    r = pl.program_id(0)
    h = _combine(p_ref, y_ref, dis_ref[...], b_ref)
    iota = lax.broadcasted_iota(jnp.int32, (512, N_GRAPHS), 1)
    oh = (batch_ref[...] == iota).astype(jnp.float32)

    @pl.when(r == 0)
    def _():
        s_ref[...] = jnp.zeros_like(s_ref)
        cnt_ref[...] = jnp.zeros_like(cnt_ref)

    sb = lax.dot_general(h, oh, (((0,), (0,)), ((), ())),
                         preferred_element_type=jnp.float32)
    for ci in range(PC):
        s_ref[ci] += sb[ci * 128:(ci + 1) * 128, :]
    cnt_ref[0:1, :] += jnp.sum(oh, axis=0, keepdims=True)


def _pool_call(part, y, dis, b_r, batch_col):
    return pl.pallas_call(
        _pool_kernel,
        grid=(20,),
        in_specs=[
            pl.BlockSpec((2, NT, 512, TW), lambda r: (0, 0, r, 0)),
            pl.BlockSpec((NT, 512, TW), lambda r: (0, r, 0)),
            pl.BlockSpec((512, 128), lambda r: (r, 0)),
            pl.BlockSpec((NT, 1, TW), lambda r: (0, 0, 0)),
            pl.BlockSpec((512, 1), lambda r: (r, 0)),
        ],
        out_specs=[
            pl.BlockSpec((PC, 128, 128), lambda r: (0, 0, 0)),
            pl.BlockSpec((8, 128), lambda r: (0, 0)),
        ],
        out_shape=[
            jax.ShapeDtypeStruct((PC, 128, N_GRAPHS), jnp.float32),
            jax.ShapeDtypeStruct((8, N_GRAPHS), jnp.float32),
        ],
    )(part, y, dis, b_r, batch_col)


def _final_kernel(s_ref, w_ref, cnt_ref, b_ref, out_ref):
    acc = jnp.zeros((N_GRAPHS, OUT_CH), jnp.float32)
    for ci in range(PC):
        acc += lax.dot_general(s_ref[ci], w_ref[ci], (((0,), (0,)), ((), ())),
                               preferred_element_type=jnp.float32)
    inv = 1.0 / jnp.maximum(cnt_ref[...], 1.0)
    out_ref[...] = acc * inv + b_ref[...]


def _final_call(S, Wl_r, cnt_col, bl_row):
    return pl.pallas_call(
        _final_kernel,
        grid=(1,),
        in_specs=[
            pl.BlockSpec((PC, 128, N_GRAPHS), lambda i: (0, 0, 0)),
            pl.BlockSpec((PC, 128, OUT_CH), lambda i: (0, 0, 0)),
            pl.BlockSpec((N_GRAPHS, 1), lambda i: (0, 0)),
            pl.BlockSpec((1, OUT_CH), lambda i: (0, 0)),
        ],
        out_specs=pl.BlockSpec((N_GRAPHS, OUT_CH), lambda i: (0, 0)),
        out_shape=jax.ShapeDtypeStruct((N_GRAPHS, OUT_CH), jnp.float32),
    )(S, Wl_r, cnt_col, bl_row)


def kernel(x, edge_index, batch, W1, b1, W2, b2, W3, b3, W_lin, b_lin):
    src = edge_index[0].astype(jnp.int32)
    dst = edge_index[1].astype(jnp.int32)
    n_e = src.shape[0]

    dummy = N_NODES + (jnp.arange(EP - n_e, dtype=jnp.int32) % (NP - N_NODES))
    src_t = jnp.concatenate([src, dummy]).reshape(NW, NBATCH, B)
    dst_t = jnp.concatenate([dst, dummy]).reshape(NW, NBATCH, B)
    x_pad = jnp.pad(x, ((0, NP - N_NODES), (0, 0)))
    batch_col = jnp.pad(batch.astype(jnp.int32), (0, NP - N_NODES),
                        constant_values=N_GRAPHS).reshape(NP, 1)
    ones128 = jnp.ones((B, 128), jnp.float32)
    zeros128 = jnp.zeros((NP, 128), jnp.float32)
    zeros_b = jnp.zeros((NP, 2, 128), jnp.bfloat16)
    b1_r = b1.reshape(NT, 1, TW)
    b2_r = b2.reshape(NT, 1, TW)
    b3_r = b3.reshape(NT, 1, TW)
    Wl_r = W_lin.reshape(PC, 128, OUT_CH)
    bl_row = b_lin.reshape(1, OUT_CH)

    degp = _deg_call(dst_t, ones128, zeros128)
    dis = _stats_call(degp)

    def to_sc(y2d):
        y3 = y2d.reshape(NT, NP, 2, 128)
        return [y3[0], y3[1]]

    y = _mm1_call(x_pad, W1, dis)
    part = _agg_call(to_sc(y), src_t, dst_t, zeros_b).reshape(2, NT, NP, TW)

    y = _combine_mm_call(part, y, dis, b1_r, W2)
    part = _agg_call(to_sc(y), src_t, dst_t, zeros_b).reshape(2, NT, NP, TW)

    y = _combine_mm_call(part, y, dis, b2_r, W3)
    part = _agg_call(to_sc(y), src_t, dst_t, zeros_b).reshape(2, NT, NP, TW)

    S, cnt = _pool_call(part, y, dis, b3_r, batch_col)
    cnt_col = cnt[0:1, :].reshape(N_GRAPHS, 1)
    return _final_call(S, Wl_r, cnt_col, bl_row)
